# baseline (speedup 1.0000x reference)
import jax
import jax.numpy as jnp
from jax.experimental import pallas as pl
from jax.experimental.pallas import tpu as pltpu

_F = 40
_KW = 13
_PK = 35
_PS = 7
_EPS = 1e-5
_LAT = 64


def _enc_kernel(x_ref, w_ref, shift_ref, pm_ref, m2_ref, y_ref):
    Bt, _one, C, T = x_ref.shape
    Cp = w_ref.shape[1] // _KW
    T1 = T - _KW + 1
    P = (T1 - _PK) // _PS + 1

    xb = x_ref[...].reshape(Bt, C, T).astype(jnp.bfloat16)
    if Cp > C:
        xb = jnp.concatenate(
            [xb, jnp.zeros((Bt, Cp - C, T), jnp.bfloat16)], axis=1)

    w = jnp.broadcast_to(w_ref[...][None], (Bt,) + w_ref.shape)

    chunks = []
    t0 = 0
    while t0 < T1:
        chunks.append((t0, min(256, T1 - t0)))
        t0 += 256
    pooled = jnp.zeros((Bt, _F, P), jnp.float32)
    for t0, tc in chunks:
        xcol = jnp.concatenate(
            [xb[:, :, t0 + k:t0 + k + tc] for k in range(_KW)],
            axis=1)
        hc = jnp.einsum("bfr,brt->bft", w, xcol,
                        preferred_element_type=jnp.float32)
        hc = hc + shift_ref[...]
        h2 = (hc * hc).astype(jnp.bfloat16)
        pooled = pooled + jax.lax.dot_general(
            h2, pm_ref[t0:t0 + tc, :], (((2,), (0,)), ((), ())),
            preferred_element_type=jnp.float32)
    logp = jnp.log(jnp.clip(pooled * (1.0 / _PK), 1e-7, 1e4))

    flat = logp.reshape(Bt, _F * P).astype(jnp.bfloat16)
    y_ref[...] = jnp.dot(flat, m2_ref[...],
                         preferred_element_type=jnp.float32)


def kernel(x, conv1_w, conv1_b, conv2_w, bn_gamma, bn_beta, bn_mean, bn_var):
    B, _, C, T = x.shape
    T1 = T - _KW + 1
    P = (T1 - _PK) // _PS + 1
    L = _F * P
    Cp = -(-C // 8) * 8

    w1_2d = conv1_w[:, 0, 0, :].astype(jnp.float32)
    w2_3d = conv2_w[:, :, :, 0].astype(jnp.float32)
    w_eff = jnp.einsum("gk,fgc->fkc", w1_2d, w2_3d)
    eff_bias = jnp.einsum("fgc,g->f", w2_3d, conv1_b.astype(jnp.float32))
    scale = bn_gamma / jnp.sqrt(bn_var + _EPS)
    shift = (eff_bias - bn_mean) * scale + bn_beta
    w_eff = w_eff * scale[:, None, None]
    w_eff = jnp.pad(w_eff, ((0, 0), (0, 0), (0, Cp - C)))
    w_eff = w_eff.reshape(_F, _KW * Cp).astype(jnp.bfloat16)
    shift3 = shift.reshape(1, _F, 1).astype(jnp.float32)

    t_idx = jnp.arange(T1)[:, None]
    p_idx = jnp.arange(P)[None, :]
    pm = ((t_idx >= _PS * p_idx) & (t_idx < _PS * p_idx + _PK)
          ).astype(jnp.bfloat16)

    n = jnp.arange(L)
    i = jnp.arange(_LAT)
    start = (i * L) // _LAT
    end = -((-(i + 1) * L) // _LAT)
    m2 = (((n[:, None] >= start[None, :]) & (n[:, None] < end[None, :])
           ).astype(jnp.float32)
          / (end - start)[None, :].astype(jnp.float32)).astype(jnp.bfloat16)

    Bt = 64 if B % 64 == 0 else (8 if B % 8 == 0 else B)
    grid = (B // Bt,)

    out = pl.pallas_call(
        _enc_kernel,
        out_shape=jax.ShapeDtypeStruct((B, _LAT), jnp.float32),
        grid=grid,
        in_specs=[
            pl.BlockSpec((Bt, 1, C, T), lambda b: (b, 0, 0, 0)),
            pl.BlockSpec((_F, _KW * Cp), lambda b: (0, 0)),
            pl.BlockSpec((1, _F, 1), lambda b: (0, 0, 0)),
            pl.BlockSpec((T1, P), lambda b: (0, 0)),
            pl.BlockSpec((L, _LAT), lambda b: (0, 0)),
        ],
        out_specs=pl.BlockSpec((Bt, _LAT), lambda b: (b, 0)),
        compiler_params=pltpu.CompilerParams(
            dimension_semantics=("parallel",)),
    )(x, w_eff, shift3, pm, m2)
    return out


# speedup vs baseline: 1.0603x; 1.0603x over previous
import jax
import jax.numpy as jnp
from jax.experimental import pallas as pl
from jax.experimental.pallas import tpu as pltpu

_F = 40
_KW = 13
_PK = 35
_PS = 7
_EPS = 1e-5
_LAT = 64


def _enc_kernel(x_ref, w_ref, shift_ref, pm_ref, m2_ref, y_ref):
    Bt, C, T = x_ref.shape
    Cp = w_ref.shape[1] // _KW
    T1 = T - _KW + 1
    P = (T1 - _PK) // _PS + 1

    xb = x_ref[...].astype(jnp.bfloat16)
    if Cp > C:
        xb = jnp.concatenate(
            [xb, jnp.zeros((Bt, Cp - C, T), jnp.bfloat16)], axis=1)

    w = jnp.broadcast_to(w_ref[...][None], (Bt,) + w_ref.shape)

    kh = 7
    chunks = []
    t0 = 0
    while t0 < T1:
        chunks.append((t0, min(256 - kh, T1 - t0)))
        t0 += 256 - kh
    pooled = jnp.zeros((Bt, _F, P), jnp.float32)
    for t0, tc in chunks:
        ext = tc + kh
        slabs = []
        for k in range(kh):
            end = t0 + k + ext
            if end <= T:
                slabs.append(xb[:, :, t0 + k:end])
            else:
                slabs.append(jnp.concatenate(
                    [xb[:, :, t0 + k:T],
                     jnp.zeros((Bt, Cp, end - T), jnp.bfloat16)], axis=2))
        xcw = jnp.concatenate(slabs, axis=1)
        ha = jnp.einsum("bfr,brt->bft", w[:, :, :kh * Cp],
                        xcw[:, :, :tc],
                        preferred_element_type=jnp.float32)
        hb = jnp.einsum("bfr,brt->bft", w[:, :, kh * Cp:],
                        xcw[:, :(_KW - kh) * Cp, :],
                        preferred_element_type=jnp.float32)
        hc = ha + hb.astype(jnp.bfloat16)[:, :, kh:kh + tc]
        hc = hc + shift_ref[...]
        h2 = (hc * hc).astype(jnp.bfloat16)
        pooled = pooled + jax.lax.dot_general(
            h2, pm_ref[t0:t0 + tc, :], (((2,), (0,)), ((), ())),
            preferred_element_type=jnp.float32)
    logp = jnp.log(jnp.clip(pooled * (1.0 / _PK), 1e-7, 1e4))

    flat = logp.reshape(Bt, _F * P).astype(jnp.bfloat16)
    y_ref[...] = jnp.dot(flat, m2_ref[...],
                         preferred_element_type=jnp.float32)


def kernel(x, conv1_w, conv1_b, conv2_w, bn_gamma, bn_beta, bn_mean, bn_var):
    B, _, C, T = x.shape
    T1 = T - _KW + 1
    P = (T1 - _PK) // _PS + 1
    L = _F * P
    Cp = -(-C // 8) * 8

    w1_2d = conv1_w[:, 0, 0, :].astype(jnp.float32)
    w2_3d = conv2_w[:, :, :, 0].astype(jnp.float32)
    w_eff = jnp.einsum("gk,fgc->fkc", w1_2d, w2_3d)
    eff_bias = jnp.einsum("fgc,g->f", w2_3d, conv1_b.astype(jnp.float32))
    scale = bn_gamma / jnp.sqrt(bn_var + _EPS)
    shift = (eff_bias - bn_mean) * scale + bn_beta
    w_eff = w_eff * scale[:, None, None]
    w_eff = jnp.pad(w_eff, ((0, 0), (0, 0), (0, Cp - C)))
    w_eff = w_eff.reshape(_F, _KW * Cp).astype(jnp.bfloat16)
    shift3 = shift.reshape(1, _F, 1).astype(jnp.float32)

    t_idx = jnp.arange(T1)[:, None]
    p_idx = jnp.arange(P)[None, :]
    pm = ((t_idx >= _PS * p_idx) & (t_idx < _PS * p_idx + _PK)
          ).astype(jnp.bfloat16)

    n = jnp.arange(L)
    i = jnp.arange(_LAT)
    start = (i * L) // _LAT
    end = -((-(i + 1) * L) // _LAT)
    m2 = (((n[:, None] >= start[None, :]) & (n[:, None] < end[None, :])
           ).astype(jnp.float32)
          / (end - start)[None, :].astype(jnp.float32)).astype(jnp.bfloat16)

    x3 = x.reshape(B, C, T)

    Bt = 64 if B % 64 == 0 else (8 if B % 8 == 0 else B)
    grid = (B // Bt,)

    out = pl.pallas_call(
        _enc_kernel,
        out_shape=jax.ShapeDtypeStruct((B, _LAT), jnp.float32),
        grid=grid,
        in_specs=[
            pl.BlockSpec((Bt, C, T), lambda b: (b, 0, 0)),
            pl.BlockSpec((_F, _KW * Cp), lambda b: (0, 0)),
            pl.BlockSpec((1, _F, 1), lambda b: (0, 0, 0)),
            pl.BlockSpec((T1, P), lambda b: (0, 0)),
            pl.BlockSpec((L, _LAT), lambda b: (0, 0)),
        ],
        out_specs=pl.BlockSpec((Bt, _LAT), lambda b: (b, 0)),
        compiler_params=pltpu.CompilerParams(
            dimension_semantics=("parallel",)),
    )(x3, w_eff, shift3, pm, m2)
    return out


# speedup vs baseline: 1.1802x; 1.1131x over previous
import jax
import jax.numpy as jnp
from jax.experimental import pallas as pl
from jax.experimental.pallas import tpu as pltpu

_F = 40
_KW = 13
_PK = 35
_PS = 7
_EPS = 1e-5
_LAT = 64


def _enc_kernel(x_ref, w_ref, shift_ref, pm_ref, m2_ref, y_ref):
    Bt, C, T = x_ref.shape
    Cp = w_ref.shape[1] // _KW
    T1 = T - _KW + 1
    P = (T1 - _PK) // _PS + 1

    xb = x_ref[...].astype(jnp.bfloat16)
    if Cp > C:
        xb = jnp.concatenate(
            [xb, jnp.zeros((Bt, Cp - C, T), jnp.bfloat16)], axis=1)

    w = jnp.broadcast_to(w_ref[...][None], (Bt,) + w_ref.shape)

    chunks = []
    t0 = 0
    while t0 < T1:
        chunks.append((t0, min(256, T1 - t0)))
        t0 += 256
    pooled = jnp.zeros((Bt, _F, P), jnp.float32)
    for t0, tc in chunks:
        xcol = jnp.concatenate(
            [xb[:, :, t0 + k:t0 + k + tc] for k in range(_KW)],
            axis=1)
        hc = jnp.einsum("bfr,brt->bft", w, xcol,
                        preferred_element_type=jnp.float32)
        hc = hc + shift_ref[...]
        h2 = (hc * hc).astype(jnp.bfloat16)
        pooled = pooled + jax.lax.dot_general(
            h2, pm_ref[t0:t0 + tc, :], (((2,), (0,)), ((), ())),
            preferred_element_type=jnp.float32)
    logp = jnp.log(jnp.clip(pooled * (1.0 / _PK), 1e-7, 1e4))

    flat = logp.reshape(Bt, _F * P).astype(jnp.bfloat16)
    y_ref[...] = jnp.dot(flat, m2_ref[...],
                         preferred_element_type=jnp.float32)


def kernel(x, conv1_w, conv1_b, conv2_w, bn_gamma, bn_beta, bn_mean, bn_var):
    B, _, C, T = x.shape
    T1 = T - _KW + 1
    P = (T1 - _PK) // _PS + 1
    L = _F * P
    Cp = -(-C // 8) * 8

    w1_2d = conv1_w[:, 0, 0, :].astype(jnp.float32)
    w2_3d = conv2_w[:, :, :, 0].astype(jnp.float32)
    w_eff = jnp.einsum("gk,fgc->fkc", w1_2d, w2_3d)
    eff_bias = jnp.einsum("fgc,g->f", w2_3d, conv1_b.astype(jnp.float32))
    scale = bn_gamma / jnp.sqrt(bn_var + _EPS)
    shift = (eff_bias - bn_mean) * scale + bn_beta
    w_eff = w_eff * scale[:, None, None]
    w_eff = jnp.pad(w_eff, ((0, 0), (0, 0), (0, Cp - C)))
    w_eff = w_eff.reshape(_F, _KW * Cp).astype(jnp.bfloat16)
    shift3 = shift.reshape(1, _F, 1).astype(jnp.float32)

    t_idx = jnp.arange(T1)[:, None]
    p_idx = jnp.arange(P)[None, :]
    pm = ((t_idx >= _PS * p_idx) & (t_idx < _PS * p_idx + _PK)
          ).astype(jnp.bfloat16)

    n = jnp.arange(L)
    i = jnp.arange(_LAT)
    start = (i * L) // _LAT
    end = -((-(i + 1) * L) // _LAT)
    m2 = (((n[:, None] >= start[None, :]) & (n[:, None] < end[None, :])
           ).astype(jnp.float32)
          / (end - start)[None, :].astype(jnp.float32)).astype(jnp.bfloat16)

    x3 = x.reshape(B, C, T)

    Bt = 64 if B % 64 == 0 else (8 if B % 8 == 0 else B)
    grid = (B // Bt,)

    out = pl.pallas_call(
        _enc_kernel,
        out_shape=jax.ShapeDtypeStruct((B, _LAT), jnp.float32),
        grid=grid,
        in_specs=[
            pl.BlockSpec((Bt, C, T), lambda b: (b, 0, 0)),
            pl.BlockSpec((_F, _KW * Cp), lambda b: (0, 0)),
            pl.BlockSpec((1, _F, 1), lambda b: (0, 0, 0)),
            pl.BlockSpec((T1, P), lambda b: (0, 0)),
            pl.BlockSpec((L, _LAT), lambda b: (0, 0)),
        ],
        out_specs=pl.BlockSpec((Bt, _LAT), lambda b: (b, 0)),
        compiler_params=pltpu.CompilerParams(
            dimension_semantics=("parallel",)),
    )(x3, w_eff, shift3, pm, m2)
    return out
